# baseline (speedup 1.0000x reference)
import functools

import jax
import jax.numpy as jnp
from jax import lax
from jax.experimental import pallas as pl
from jax.experimental.pallas import tpu as pltpu
from jax.experimental.pallas import tpu_sc as plsc

_B = 1024
_HID = 32
_LAMBDA = 1e-05

_NC = 2
_NS = 16
_NW = _NC * _NS
_BPW = _B // _NW

_BN = 2048


def _gather_body(user_emb, item_emb, u_idx, i_idx, user_bias, item_bias,
                 ue_out, ie_out, ub_out, ib_out,
                 uidx_v, iidx_v, ue_v, ie_v, ub_v, ib_v, s0, s1, s2, s3):
    wid = lax.axis_index("s") * _NC + lax.axis_index("c")
    base = wid * _BPW
    pltpu.sync_copy(u_idx.at[pl.ds(base, _BPW)], uidx_v)
    pltpu.sync_copy(i_idx.at[pl.ds(base, _BPW)], iidx_v)
    c0 = pltpu.async_copy(user_emb.at[uidx_v], ue_v, s0)
    c1 = pltpu.async_copy(item_emb.at[iidx_v], ie_v, s1)
    c2 = pltpu.async_copy(user_bias.at[uidx_v], ub_v, s2)
    c3 = pltpu.async_copy(item_bias.at[iidx_v], ib_v, s3)
    c0.wait()
    c1.wait()
    c2.wait()
    c3.wait()
    pltpu.sync_copy(ue_v, ue_out.at[pl.ds(base, _BPW)])
    pltpu.sync_copy(ie_v, ie_out.at[pl.ds(base, _BPW)])
    pltpu.sync_copy(ub_v, ub_out.at[pl.ds(base, _BPW)])
    pltpu.sync_copy(ib_v, ib_out.at[pl.ds(base, _BPW)])


def _sc_gather(user_emb, item_emb, u, i, user_bias, item_bias):
    mesh = plsc.VectorSubcoreMesh(
        core_axis_name="c", subcore_axis_name="s",
        num_cores=_NC, num_subcores=_NS)
    f = pl.kernel(
        _gather_body,
        out_type=(
            jax.ShapeDtypeStruct((_B, _HID), jnp.float32),
            jax.ShapeDtypeStruct((_B, _HID), jnp.float32),
            jax.ShapeDtypeStruct((_B,), jnp.float32),
            jax.ShapeDtypeStruct((_B,), jnp.float32),
        ),
        mesh=mesh,
        scratch_types=[
            pltpu.VMEM((_BPW,), jnp.int32),
            pltpu.VMEM((_BPW,), jnp.int32),
            pltpu.VMEM((_BPW, _HID), jnp.float32),
            pltpu.VMEM((_BPW, _HID), jnp.float32),
            pltpu.VMEM((_BPW,), jnp.float32),
            pltpu.VMEM((_BPW,), jnp.float32),
            pltpu.SemaphoreType.DMA,
            pltpu.SemaphoreType.DMA,
            pltpu.SemaphoreType.DMA,
            pltpu.SemaphoreType.DMA,
        ],
        compiler_params=pltpu.CompilerParams(use_tc_tiling_on_sc=False),
    )
    return f(user_emb, item_emb, u, i, user_bias, item_bias)


def _score_body(ue_ref, ie_ref, ub_ref, ib_ref, y_ref, gb_ref,
                iemb_ref, ibias_ref,
                s_ref, score_ref, loss_ref, diff_ref):
    pid = pl.program_id(0)
    ue = ue_ref[...]
    blk = iemb_ref[...]
    sc = lax.dot_general(ue, blk, (((1,), (1,)), ((), ())),
                         preferred_element_type=jnp.float32)
    score_ref[...] = sc + ibias_ref[...][None, :]

    @pl.when(pid == 0)
    def _small():
        ie = ie_ref[...]
        ub = ub_ref[...]
        ib = ib_ref[...]
        s = jnp.sum(ue * ie, axis=1) + ub + ib + gb_ref[0, 0]
        d = s - y_ref[...]
        s_ref[...] = s
        diff_ref[...] = d
        l2 = (jnp.mean(ue * ue) + jnp.mean(ie * ie)
              + jnp.mean(ub * ub) + jnp.mean(ib * ib))
        loss_ref[0, 0] = jnp.mean(d * d) + _LAMBDA * l2


def _tc_score(ue, ie, ub, ib, y, gb2d, item_emb, item_bias):
    n_item = item_emb.shape[0]
    grid = (pl.cdiv(n_item, _BN),)
    return pl.pallas_call(
        _score_body,
        grid=grid,
        in_specs=[
            pl.BlockSpec((_B, _HID), lambda n: (0, 0)),
            pl.BlockSpec((_B, _HID), lambda n: (0, 0)),
            pl.BlockSpec((_B,), lambda n: (0,)),
            pl.BlockSpec((_B,), lambda n: (0,)),
            pl.BlockSpec((_B,), lambda n: (0,)),
            pl.BlockSpec((1, 1), lambda n: (0, 0),
                         memory_space=pltpu.SMEM),
            pl.BlockSpec((_BN, _HID), lambda n: (n, 0)),
            pl.BlockSpec((_BN,), lambda n: (n,)),
        ],
        out_specs=[
            pl.BlockSpec((_B,), lambda n: (0,)),
            pl.BlockSpec((_B, _BN), lambda n: (0, n)),
            pl.BlockSpec((1, 1), lambda n: (0, 0),
                         memory_space=pltpu.SMEM),
            pl.BlockSpec((_B,), lambda n: (0,)),
        ],
        out_shape=[
            jax.ShapeDtypeStruct((_B,), jnp.float32),
            jax.ShapeDtypeStruct((_B, n_item), jnp.float32),
            jax.ShapeDtypeStruct((1, 1), jnp.float32),
            jax.ShapeDtypeStruct((_B,), jnp.float32),
        ],
        compiler_params=pltpu.CompilerParams(
            dimension_semantics=("arbitrary",),
        ),
    )(ue, ie, ub, ib, y, gb2d, item_emb, item_bias)


def kernel(u, i, y, user_emb, item_emb, user_bias, item_bias, global_bias):
    ue, ie, ub, ib = _sc_gather(user_emb, item_emb, u, i,
                                user_bias, item_bias)
    gb2d = jnp.reshape(global_bias, (1, 1)).astype(jnp.float32)
    s, score, loss, diff = _tc_score(ue, ie, ub, ib, y, gb2d,
                                     item_emb, item_bias)
    return s, score, jnp.reshape(loss, ()), diff


# speedup vs baseline: 5.7079x; 5.7079x over previous
import jax
import jax.numpy as jnp
from jax import lax
from jax.experimental import pallas as pl
from jax.experimental.pallas import tpu as pltpu
from jax.experimental.pallas import tpu_sc as plsc

_B = 1024
_HID = 32
_LAMBDA = 1e-05
_NU = 1000000
_NI = 100000

_NC = 2
_NS = 16
_NW = _NC * _NS
_BPW = _B // _NW

_CU_MAX = _NU // 128 - 1
_U_TAIL_OFF = (_CU_MAX + 1) * 128
_U_TAIL_W = _NU - _U_TAIL_OFF
_CI_MAX = _NI // 128 - 1
_I_TAIL_OFF = (_CI_MAX + 1) * 128
_I_TAIL_W = _NI - _I_TAIL_OFF

_RING = 4

_BN = 2048


def _emb_gather_body(et_hbm, eti_hbm, u_hbm, i_hbm,
                     ue_out, ie_out,
                     uidx_v, iidx_v, ue_loc, ie_loc, utail, itail,
                     bufs_u, bufs_i, sems_u, sems_i, tsem):
    wid = lax.axis_index("s") * _NC + lax.axis_index("c")
    base = wid * _BPW
    pltpu.sync_copy(u_hbm.at[pl.ds(base, _BPW)], uidx_v)
    pltpu.sync_copy(i_hbm.at[pl.ds(base, _BPW)], iidx_v)
    ct = pltpu.async_copy(et_hbm.at[:, pl.ds(_U_TAIL_OFF, _U_TAIL_W)],
                          utail, tsem)
    ct.wait()
    ct2 = pltpu.async_copy(eti_hbm.at[:, pl.ds(_I_TAIL_OFF, _I_TAIL_W)],
                           itail, tsem)
    ct2.wait()

    iota16 = lax.iota(jnp.int32, 16)

    def issue(table, idx_v, bufs, sems, cmax, j):
        chunk = idx_v[pl.ds((j // 16) * 16, 16)]
        xj = chunk[j % 16]
        c = jnp.minimum(xj // 128, cmax)
        off = pl.multiple_of(c * 128, 128)
        slot = j % _RING
        return pltpu.async_copy(table.at[:, pl.ds(off, 128)],
                                bufs[slot], sems[slot])

    def extract(idx_v, bufs, tail, loc, cmax, toff, tw, j):
        chunk = idx_v[pl.ds((j // 16) * 16, 16)]
        xj = chunk[j % 16]
        c = jnp.minimum(xj // 128, cmax)
        lane_m = jnp.minimum(xj - c * 128, 127)
        lane_t = jnp.clip(xj - toff, 0, tw - 1)
        in_tail = jnp.full((16,), xj >= toff, jnp.bool_)
        slot = j % _RING
        for h in range(2):
            rows = iota16 + h * 16
            vm = plsc.load_gather(bufs[slot],
                                  [rows, jnp.full((16,), lane_m, jnp.int32)])
            vt = plsc.load_gather(tail,
                                  [rows, jnp.full((16,), lane_t, jnp.int32)])
            loc[j, pl.ds(h * 16, 16)] = jnp.where(in_tail, vt, vm)

    pend_u = [None] * _RING
    pend_i = [None] * _RING
    for j in range(_RING):
        pend_u[j] = issue(et_hbm, uidx_v, bufs_u, sems_u, _CU_MAX, j)
        pend_i[j] = issue(eti_hbm, iidx_v, bufs_i, sems_i, _CI_MAX, j)
    for j in range(_BPW):
        pend_u[j % _RING].wait()
        extract(uidx_v, bufs_u, utail, ue_loc, _CU_MAX,
                _U_TAIL_OFF, _U_TAIL_W, j)
        pend_i[j % _RING].wait()
        extract(iidx_v, bufs_i, itail, ie_loc, _CI_MAX,
                _I_TAIL_OFF, _I_TAIL_W, j)
        nj = j + _RING
        if nj < _BPW:
            pend_u[nj % _RING] = issue(et_hbm, uidx_v, bufs_u, sems_u,
                                       _CU_MAX, nj)
            pend_i[nj % _RING] = issue(eti_hbm, iidx_v, bufs_i, sems_i,
                                       _CI_MAX, nj)

    pltpu.sync_copy(ue_loc, ue_out.at[pl.ds(base, _BPW), :])
    pltpu.sync_copy(ie_loc, ie_out.at[pl.ds(base, _BPW), :])


def _sc_emb_gather(et, eti, u, i):
    mesh = plsc.VectorSubcoreMesh(
        core_axis_name="c", subcore_axis_name="s",
        num_cores=_NC, num_subcores=_NS)
    f = pl.kernel(
        _emb_gather_body,
        out_type=(
            jax.ShapeDtypeStruct((_B, _HID), jnp.float32),
            jax.ShapeDtypeStruct((_B, _HID), jnp.float32),
        ),
        mesh=mesh,
        scratch_types=[
            pltpu.VMEM((_BPW,), jnp.int32),
            pltpu.VMEM((_BPW,), jnp.int32),
            pltpu.VMEM((_BPW, _HID), jnp.float32),
            pltpu.VMEM((_BPW, _HID), jnp.float32),
            pltpu.VMEM((_HID, _U_TAIL_W), jnp.float32),
            pltpu.VMEM((_HID, _I_TAIL_W), jnp.float32),
            [pltpu.VMEM((_HID, 128), jnp.float32)] * _RING,
            [pltpu.VMEM((_HID, 128), jnp.float32)] * _RING,
            [pltpu.SemaphoreType.DMA] * _RING,
            [pltpu.SemaphoreType.DMA] * _RING,
            pltpu.SemaphoreType.DMA,
        ],
        compiler_params=pltpu.CompilerParams(use_tc_tiling_on_sc=True,
                                             needs_layout_passes=False),
    )
    return f(et, eti, u, i)


def _bias_gather_body(ub_hbm, ib_hbm, u_hbm, i_hbm,
                      ub_out, ib_out,
                      uidx_v, iidx_v, ub_v, ib_v, s0, s1):
    wid = lax.axis_index("s") * _NC + lax.axis_index("c")
    base = wid * _BPW
    pltpu.sync_copy(u_hbm.at[pl.ds(base, _BPW)], uidx_v)
    pltpu.sync_copy(i_hbm.at[pl.ds(base, _BPW)], iidx_v)
    c0 = pltpu.async_copy(ub_hbm.at[uidx_v], ub_v, s0)
    c1 = pltpu.async_copy(ib_hbm.at[iidx_v], ib_v, s1)
    c0.wait()
    c1.wait()
    pltpu.sync_copy(ub_v, ub_out.at[pl.ds(base, _BPW)])
    pltpu.sync_copy(ib_v, ib_out.at[pl.ds(base, _BPW)])


def _sc_bias_gather(user_bias, item_bias, u, i):
    mesh = plsc.VectorSubcoreMesh(
        core_axis_name="c", subcore_axis_name="s",
        num_cores=_NC, num_subcores=_NS)
    f = pl.kernel(
        _bias_gather_body,
        out_type=(
            jax.ShapeDtypeStruct((_B,), jnp.float32),
            jax.ShapeDtypeStruct((_B,), jnp.float32),
        ),
        mesh=mesh,
        scratch_types=[
            pltpu.VMEM((_BPW,), jnp.int32),
            pltpu.VMEM((_BPW,), jnp.int32),
            pltpu.VMEM((_BPW,), jnp.float32),
            pltpu.VMEM((_BPW,), jnp.float32),
            pltpu.SemaphoreType.DMA,
            pltpu.SemaphoreType.DMA,
        ],
        compiler_params=pltpu.CompilerParams(use_tc_tiling_on_sc=False),
    )
    return f(user_bias, item_bias, u, i)


def _score_body(ue_ref, ie_ref, ub_ref, ib_ref, y_ref, gb_ref,
                eti_ref, ibias_ref,
                s_ref, score_ref, loss_ref, diff_ref,
                at_ref):
    pid = pl.program_id(0)

    @pl.when(pid == 0)
    def _prep():
        at_ref[...] = ue_ref[...].T

    sc = lax.dot_general(eti_ref[...], at_ref[...],
                         (((0,), (0,)), ((), ())),
                         preferred_element_type=jnp.float32)
    score_ref[...] = sc + ibias_ref[...][:, None]

    @pl.when(pid == 0)
    def _small():
        ue = ue_ref[...]
        ie = ie_ref[...]
        ub = ub_ref[...]
        ib = ib_ref[...]
        s = jnp.sum(ue * ie, axis=1) + ub + ib + gb_ref[0, 0]
        d = s - y_ref[...]
        s_ref[...] = s
        diff_ref[...] = d
        l2 = (jnp.mean(ue * ue) + jnp.mean(ie * ie)
              + jnp.mean(ub * ub) + jnp.mean(ib * ib))
        loss_ref[0, 0] = jnp.mean(d * d) + _LAMBDA * l2


def _tc_score(ue, ie, ub, ib, y, gb2d, eti, item_bias):
    grid = (pl.cdiv(_NI, _BN),)
    return pl.pallas_call(
        _score_body,
        grid=grid,
        in_specs=[
            pl.BlockSpec((_B, _HID), lambda n: (0, 0)),
            pl.BlockSpec((_B, _HID), lambda n: (0, 0)),
            pl.BlockSpec((_B,), lambda n: (0,)),
            pl.BlockSpec((_B,), lambda n: (0,)),
            pl.BlockSpec((_B,), lambda n: (0,)),
            pl.BlockSpec((1, 1), lambda n: (0, 0),
                         memory_space=pltpu.SMEM),
            pl.BlockSpec((_HID, _BN), lambda n: (0, n)),
            pl.BlockSpec((_BN,), lambda n: (n,)),
        ],
        out_specs=[
            pl.BlockSpec((_B,), lambda n: (0,)),
            pl.BlockSpec((_BN, _B), lambda n: (n, 0)),
            pl.BlockSpec((1, 1), lambda n: (0, 0),
                         memory_space=pltpu.SMEM),
            pl.BlockSpec((_B,), lambda n: (0,)),
        ],
        out_shape=[
            jax.ShapeDtypeStruct((_B,), jnp.float32),
            jax.ShapeDtypeStruct((_NI, _B), jnp.float32),
            jax.ShapeDtypeStruct((1, 1), jnp.float32),
            jax.ShapeDtypeStruct((_B,), jnp.float32),
        ],
        scratch_shapes=[pltpu.VMEM((_HID, _B), jnp.float32)],
        compiler_params=pltpu.CompilerParams(
            dimension_semantics=("arbitrary",),
        ),
    )(ue, ie, ub, ib, y, gb2d, eti, item_bias)


def kernel(u, i, y, user_emb, item_emb, user_bias, item_bias, global_bias):
    et = user_emb.T
    eti = item_emb.T
    ue, ie = _sc_emb_gather(et, eti, u, i)
    ub, ib = _sc_bias_gather(user_bias, item_bias, u, i)
    gb2d = jnp.reshape(global_bias, (1, 1)).astype(jnp.float32)
    s, score_t, loss, diff = _tc_score(ue, ie, ub, ib, y, gb2d,
                                       eti, item_bias)
    return s, score_t.T, jnp.reshape(loss, ()), diff


# speedup vs baseline: 5.7086x; 1.0001x over previous
import jax
import jax.numpy as jnp
from jax import lax
from jax.experimental import pallas as pl
from jax.experimental.pallas import tpu as pltpu
from jax.experimental.pallas import tpu_sc as plsc

_B = 1024
_HID = 32
_LAMBDA = 1e-05
_NU = 1000000
_NI = 100000

_NC = 2
_NS = 16
_NW = _NC * _NS
_BPW = _B // _NW

_CU_MAX = _NU // 128 - 1
_U_TAIL_OFF = (_CU_MAX + 1) * 128
_U_TAIL_W = _NU - _U_TAIL_OFF
_CI_MAX = _NI // 128 - 1
_I_TAIL_OFF = (_CI_MAX + 1) * 128
_I_TAIL_W = _NI - _I_TAIL_OFF

_RING = 4

_BN = 4096


def _emb_gather_body(et_hbm, eti_hbm, u_hbm, i_hbm,
                     ue_out, ie_out,
                     uidx_v, iidx_v, ue_loc, ie_loc, utail, itail,
                     bufs_u, bufs_i, sems_u, sems_i, tsem):
    wid = lax.axis_index("s") * _NC + lax.axis_index("c")
    base = wid * _BPW
    pltpu.sync_copy(u_hbm.at[pl.ds(base, _BPW)], uidx_v)
    pltpu.sync_copy(i_hbm.at[pl.ds(base, _BPW)], iidx_v)
    ct = pltpu.async_copy(et_hbm.at[:, pl.ds(_U_TAIL_OFF, _U_TAIL_W)],
                          utail, tsem)
    ct.wait()
    ct2 = pltpu.async_copy(eti_hbm.at[:, pl.ds(_I_TAIL_OFF, _I_TAIL_W)],
                           itail, tsem)
    ct2.wait()

    iota16 = lax.iota(jnp.int32, 16)

    def issue(table, idx_v, bufs, sems, cmax, j):
        chunk = idx_v[pl.ds((j // 16) * 16, 16)]
        xj = chunk[j % 16]
        c = jnp.minimum(xj // 128, cmax)
        off = pl.multiple_of(c * 128, 128)
        slot = j % _RING
        return pltpu.async_copy(table.at[:, pl.ds(off, 128)],
                                bufs[slot], sems[slot])

    def extract(idx_v, bufs, tail, loc, cmax, toff, tw, j):
        chunk = idx_v[pl.ds((j // 16) * 16, 16)]
        xj = chunk[j % 16]
        c = jnp.minimum(xj // 128, cmax)
        lane_m = jnp.minimum(xj - c * 128, 127)
        lane_t = jnp.clip(xj - toff, 0, tw - 1)
        in_tail = jnp.full((16,), xj >= toff, jnp.bool_)
        slot = j % _RING
        for h in range(2):
            rows = iota16 + h * 16
            vm = plsc.load_gather(bufs[slot],
                                  [rows, jnp.full((16,), lane_m, jnp.int32)])
            vt = plsc.load_gather(tail,
                                  [rows, jnp.full((16,), lane_t, jnp.int32)])
            loc[j, pl.ds(h * 16, 16)] = jnp.where(in_tail, vt, vm)

    pend_u = [None] * _RING
    pend_i = [None] * _RING
    for j in range(_RING):
        pend_u[j] = issue(et_hbm, uidx_v, bufs_u, sems_u, _CU_MAX, j)
        pend_i[j] = issue(eti_hbm, iidx_v, bufs_i, sems_i, _CI_MAX, j)
    for j in range(_BPW):
        pend_u[j % _RING].wait()
        extract(uidx_v, bufs_u, utail, ue_loc, _CU_MAX,
                _U_TAIL_OFF, _U_TAIL_W, j)
        pend_i[j % _RING].wait()
        extract(iidx_v, bufs_i, itail, ie_loc, _CI_MAX,
                _I_TAIL_OFF, _I_TAIL_W, j)
        nj = j + _RING
        if nj < _BPW:
            pend_u[nj % _RING] = issue(et_hbm, uidx_v, bufs_u, sems_u,
                                       _CU_MAX, nj)
            pend_i[nj % _RING] = issue(eti_hbm, iidx_v, bufs_i, sems_i,
                                       _CI_MAX, nj)

    pltpu.sync_copy(ue_loc, ue_out.at[pl.ds(base, _BPW), :])
    pltpu.sync_copy(ie_loc, ie_out.at[pl.ds(base, _BPW), :])


def _sc_emb_gather(et, eti, u, i):
    mesh = plsc.VectorSubcoreMesh(
        core_axis_name="c", subcore_axis_name="s",
        num_cores=_NC, num_subcores=_NS)
    f = pl.kernel(
        _emb_gather_body,
        out_type=(
            jax.ShapeDtypeStruct((_B, _HID), jnp.float32),
            jax.ShapeDtypeStruct((_B, _HID), jnp.float32),
        ),
        mesh=mesh,
        scratch_types=[
            pltpu.VMEM((_BPW,), jnp.int32),
            pltpu.VMEM((_BPW,), jnp.int32),
            pltpu.VMEM((_BPW, _HID), jnp.float32),
            pltpu.VMEM((_BPW, _HID), jnp.float32),
            pltpu.VMEM((_HID, _U_TAIL_W), jnp.float32),
            pltpu.VMEM((_HID, _I_TAIL_W), jnp.float32),
            [pltpu.VMEM((_HID, 128), jnp.float32)] * _RING,
            [pltpu.VMEM((_HID, 128), jnp.float32)] * _RING,
            [pltpu.SemaphoreType.DMA] * _RING,
            [pltpu.SemaphoreType.DMA] * _RING,
            pltpu.SemaphoreType.DMA,
        ],
        compiler_params=pltpu.CompilerParams(use_tc_tiling_on_sc=True,
                                             needs_layout_passes=False),
    )
    return f(et, eti, u, i)


def _bias_gather_body(ub_hbm, ib_hbm, u_hbm, i_hbm,
                      ub_out, ib_out,
                      uidx_v, iidx_v, ub_v, ib_v, s0, s1):
    wid = lax.axis_index("s") * _NC + lax.axis_index("c")
    base = wid * _BPW
    pltpu.sync_copy(u_hbm.at[pl.ds(base, _BPW)], uidx_v)
    pltpu.sync_copy(i_hbm.at[pl.ds(base, _BPW)], iidx_v)
    c0 = pltpu.async_copy(ub_hbm.at[uidx_v], ub_v, s0)
    c1 = pltpu.async_copy(ib_hbm.at[iidx_v], ib_v, s1)
    c0.wait()
    c1.wait()
    pltpu.sync_copy(ub_v, ub_out.at[pl.ds(base, _BPW)])
    pltpu.sync_copy(ib_v, ib_out.at[pl.ds(base, _BPW)])


def _sc_bias_gather(user_bias, item_bias, u, i):
    mesh = plsc.VectorSubcoreMesh(
        core_axis_name="c", subcore_axis_name="s",
        num_cores=_NC, num_subcores=_NS)
    f = pl.kernel(
        _bias_gather_body,
        out_type=(
            jax.ShapeDtypeStruct((_B,), jnp.float32),
            jax.ShapeDtypeStruct((_B,), jnp.float32),
        ),
        mesh=mesh,
        scratch_types=[
            pltpu.VMEM((_BPW,), jnp.int32),
            pltpu.VMEM((_BPW,), jnp.int32),
            pltpu.VMEM((_BPW,), jnp.float32),
            pltpu.VMEM((_BPW,), jnp.float32),
            pltpu.SemaphoreType.DMA,
            pltpu.SemaphoreType.DMA,
        ],
        compiler_params=pltpu.CompilerParams(use_tc_tiling_on_sc=False),
    )
    return f(user_bias, item_bias, u, i)


def _score_body(ue_ref, ie_ref, ub_ref, ib_ref, y_ref, gb_ref,
                eti_ref, ibias_ref,
                s_ref, score_ref, loss_ref, diff_ref,
                at_ref):
    pid = pl.program_id(0)

    @pl.when(pid == 0)
    def _prep():
        at_ref[...] = ue_ref[...].T

    sc = lax.dot_general(eti_ref[...], at_ref[...],
                         (((0,), (0,)), ((), ())),
                         preferred_element_type=jnp.float32)
    score_ref[...] = sc + ibias_ref[...][:, None]

    @pl.when(pid == 0)
    def _small():
        ue = ue_ref[...]
        ie = ie_ref[...]
        ub = ub_ref[...]
        ib = ib_ref[...]
        s = jnp.sum(ue * ie, axis=1) + ub + ib + gb_ref[0, 0]
        d = s - y_ref[...]
        s_ref[...] = s
        diff_ref[...] = d
        l2 = (jnp.mean(ue * ue) + jnp.mean(ie * ie)
              + jnp.mean(ub * ub) + jnp.mean(ib * ib))
        loss_ref[0, 0] = jnp.mean(d * d) + _LAMBDA * l2


def _tc_score(ue, ie, ub, ib, y, gb2d, eti, item_bias):
    grid = (pl.cdiv(_NI, _BN),)
    return pl.pallas_call(
        _score_body,
        grid=grid,
        in_specs=[
            pl.BlockSpec((_B, _HID), lambda n: (0, 0)),
            pl.BlockSpec((_B, _HID), lambda n: (0, 0)),
            pl.BlockSpec((_B,), lambda n: (0,)),
            pl.BlockSpec((_B,), lambda n: (0,)),
            pl.BlockSpec((_B,), lambda n: (0,)),
            pl.BlockSpec((1, 1), lambda n: (0, 0),
                         memory_space=pltpu.SMEM),
            pl.BlockSpec((_HID, _BN), lambda n: (0, n)),
            pl.BlockSpec((_BN,), lambda n: (n,)),
        ],
        out_specs=[
            pl.BlockSpec((_B,), lambda n: (0,)),
            pl.BlockSpec((_BN, _B), lambda n: (n, 0)),
            pl.BlockSpec((1, 1), lambda n: (0, 0),
                         memory_space=pltpu.SMEM),
            pl.BlockSpec((_B,), lambda n: (0,)),
        ],
        out_shape=[
            jax.ShapeDtypeStruct((_B,), jnp.float32),
            jax.ShapeDtypeStruct((_NI, _B), jnp.float32),
            jax.ShapeDtypeStruct((1, 1), jnp.float32),
            jax.ShapeDtypeStruct((_B,), jnp.float32),
        ],
        scratch_shapes=[pltpu.VMEM((_HID, _B), jnp.float32)],
        compiler_params=pltpu.CompilerParams(
            dimension_semantics=("arbitrary",),
        ),
    )(ue, ie, ub, ib, y, gb2d, eti, item_bias)


def kernel(u, i, y, user_emb, item_emb, user_bias, item_bias, global_bias):
    et = user_emb.T
    eti = item_emb.T
    ue, ie = _sc_emb_gather(et, eti, u, i)
    ub, ib = _sc_bias_gather(user_bias, item_bias, u, i)
    gb2d = jnp.reshape(global_bias, (1, 1)).astype(jnp.float32)
    s, score_t, loss, diff = _tc_score(ue, ie, ub, ib, y, gb2d,
                                       eti, item_bias)
    return s, score_t.T, jnp.reshape(loss, ()), diff


# speedup vs baseline: 5.8973x; 1.0331x over previous
import jax
import jax.numpy as jnp
from jax import lax
from jax.experimental import pallas as pl
from jax.experimental.pallas import tpu as pltpu
from jax.experimental.pallas import tpu_sc as plsc

_B = 1024
_HID = 32
_LAMBDA = 1e-05
_NU = 1000000
_NI = 100000

_NC = 2
_NS = 16
_NW = _NC * _NS
_BPW = _B // _NW

_CU_MAX = _NU // 128 - 1
_U_TAIL_OFF = (_CU_MAX + 1) * 128
_U_TAIL_W = _NU - _U_TAIL_OFF
_CI_MAX = _NI // 128 - 1
_I_TAIL_OFF = (_CI_MAX + 1) * 128
_I_TAIL_W = _NI - _I_TAIL_OFF

_RING = 8

_BN = 2048


def _emb_gather_body(et_hbm, eti_hbm, u_hbm, i_hbm, ub_hbm, ib_hbm,
                     ue_out, ie_out, ub_out, ib_out,
                     uidx_v, iidx_v, ue_loc, ie_loc, utail, itail,
                     ub_v, ib_v, bsem,
                     bufs_u, bufs_i, sems_u, sems_i, tsem):
    wid = lax.axis_index("s") * _NC + lax.axis_index("c")
    base = wid * _BPW
    pltpu.sync_copy(u_hbm.at[pl.ds(base, _BPW)], uidx_v)
    pltpu.sync_copy(i_hbm.at[pl.ds(base, _BPW)], iidx_v)
    cb0 = pltpu.async_copy(ub_hbm.at[uidx_v], ub_v, bsem)
    cb1 = pltpu.async_copy(ib_hbm.at[iidx_v], ib_v, bsem)
    ct = pltpu.async_copy(et_hbm.at[:, pl.ds(_U_TAIL_OFF, _U_TAIL_W)],
                          utail, tsem)
    ct.wait()
    ct2 = pltpu.async_copy(eti_hbm.at[:, pl.ds(_I_TAIL_OFF, _I_TAIL_W)],
                           itail, tsem)
    ct2.wait()

    iota16 = lax.iota(jnp.int32, 16)

    def issue(table, idx_v, bufs, sems, cmax, j):
        chunk = idx_v[pl.ds((j // 16) * 16, 16)]
        xj = chunk[j % 16]
        c = jnp.minimum(xj // 128, cmax)
        off = pl.multiple_of(c * 128, 128)
        slot = j % _RING
        return pltpu.async_copy(table.at[:, pl.ds(off, 128)],
                                bufs[slot], sems[slot])

    def extract(idx_v, bufs, tail, loc, cmax, toff, tw, j):
        chunk = idx_v[pl.ds((j // 16) * 16, 16)]
        xj = chunk[j % 16]
        c = jnp.minimum(xj // 128, cmax)
        lane_m = jnp.minimum(xj - c * 128, 127)
        lane_t = jnp.clip(xj - toff, 0, tw - 1)
        in_tail = jnp.full((16,), xj >= toff, jnp.bool_)
        slot = j % _RING
        for h in range(2):
            rows = iota16 + h * 16
            vm = plsc.load_gather(bufs[slot],
                                  [rows, jnp.full((16,), lane_m, jnp.int32)])
            vt = plsc.load_gather(tail,
                                  [rows, jnp.full((16,), lane_t, jnp.int32)])
            loc[j, pl.ds(h * 16, 16)] = jnp.where(in_tail, vt, vm)

    pend_u = [None] * _RING
    pend_i = [None] * _RING
    for j in range(_RING):
        pend_u[j] = issue(et_hbm, uidx_v, bufs_u, sems_u, _CU_MAX, j)
        pend_i[j] = issue(eti_hbm, iidx_v, bufs_i, sems_i, _CI_MAX, j)
    for j in range(_BPW):
        pend_u[j % _RING].wait()
        extract(uidx_v, bufs_u, utail, ue_loc, _CU_MAX,
                _U_TAIL_OFF, _U_TAIL_W, j)
        pend_i[j % _RING].wait()
        extract(iidx_v, bufs_i, itail, ie_loc, _CI_MAX,
                _I_TAIL_OFF, _I_TAIL_W, j)
        nj = j + _RING
        if nj < _BPW:
            pend_u[nj % _RING] = issue(et_hbm, uidx_v, bufs_u, sems_u,
                                       _CU_MAX, nj)
            pend_i[nj % _RING] = issue(eti_hbm, iidx_v, bufs_i, sems_i,
                                       _CI_MAX, nj)

    pltpu.sync_copy(ue_loc, ue_out.at[pl.ds(base, _BPW), :])
    pltpu.sync_copy(ie_loc, ie_out.at[pl.ds(base, _BPW), :])
    cb0.wait()
    cb1.wait()
    pltpu.sync_copy(ub_v, ub_out.at[pl.ds(base, _BPW)])
    pltpu.sync_copy(ib_v, ib_out.at[pl.ds(base, _BPW)])


def _sc_emb_gather(et, eti, u, i, user_bias, item_bias):
    mesh = plsc.VectorSubcoreMesh(
        core_axis_name="c", subcore_axis_name="s",
        num_cores=_NC, num_subcores=_NS)
    f = pl.kernel(
        _emb_gather_body,
        out_type=(
            jax.ShapeDtypeStruct((_B, _HID), jnp.float32),
            jax.ShapeDtypeStruct((_B, _HID), jnp.float32),
            jax.ShapeDtypeStruct((_B,), jnp.float32),
            jax.ShapeDtypeStruct((_B,), jnp.float32),
        ),
        mesh=mesh,
        scratch_types=[
            pltpu.VMEM((_BPW,), jnp.int32),
            pltpu.VMEM((_BPW,), jnp.int32),
            pltpu.VMEM((_BPW, _HID), jnp.float32),
            pltpu.VMEM((_BPW, _HID), jnp.float32),
            pltpu.VMEM((_HID, _U_TAIL_W), jnp.float32),
            pltpu.VMEM((_HID, _I_TAIL_W), jnp.float32),
            pltpu.VMEM((_BPW,), jnp.float32),
            pltpu.VMEM((_BPW,), jnp.float32),
            pltpu.SemaphoreType.DMA,
            [pltpu.VMEM((_HID, 128), jnp.float32)] * _RING,
            [pltpu.VMEM((_HID, 128), jnp.float32)] * _RING,
            [pltpu.SemaphoreType.DMA] * _RING,
            [pltpu.SemaphoreType.DMA] * _RING,
            pltpu.SemaphoreType.DMA,
        ],
        compiler_params=pltpu.CompilerParams(use_tc_tiling_on_sc=True,
                                             needs_layout_passes=False),
    )
    return f(et, eti, u, i, user_bias, item_bias)


def _score_body(ue_ref, ie_ref, ub_ref, ib_ref, y_ref, gb_ref,
                eti_ref, ibias_ref,
                s_ref, score_ref, loss_ref, diff_ref,
                at_ref):
    pid = pl.program_id(0)

    @pl.when(pid == 0)
    def _prep():
        at_ref[...] = ue_ref[...].T

    sc = lax.dot_general(eti_ref[...], at_ref[...],
                         (((0,), (0,)), ((), ())),
                         preferred_element_type=jnp.float32)
    score_ref[...] = sc + ibias_ref[...][:, None]

    @pl.when(pid == 0)
    def _small():
        ue = ue_ref[...]
        ie = ie_ref[...]
        ub = ub_ref[...]
        ib = ib_ref[...]
        s = jnp.sum(ue * ie, axis=1) + ub + ib + gb_ref[0, 0]
        d = s - y_ref[...]
        s_ref[...] = s
        diff_ref[...] = d
        l2 = (jnp.mean(ue * ue) + jnp.mean(ie * ie)
              + jnp.mean(ub * ub) + jnp.mean(ib * ib))
        loss_ref[0, 0] = jnp.mean(d * d) + _LAMBDA * l2


def _tc_score(ue, ie, ub, ib, y, gb2d, eti, item_bias):
    grid = (pl.cdiv(_NI, _BN),)
    return pl.pallas_call(
        _score_body,
        grid=grid,
        in_specs=[
            pl.BlockSpec((_B, _HID), lambda n: (0, 0)),
            pl.BlockSpec((_B, _HID), lambda n: (0, 0)),
            pl.BlockSpec((_B,), lambda n: (0,)),
            pl.BlockSpec((_B,), lambda n: (0,)),
            pl.BlockSpec((_B,), lambda n: (0,)),
            pl.BlockSpec((1, 1), lambda n: (0, 0),
                         memory_space=pltpu.SMEM),
            pl.BlockSpec((_HID, _BN), lambda n: (0, n)),
            pl.BlockSpec((_BN,), lambda n: (n,)),
        ],
        out_specs=[
            pl.BlockSpec((_B,), lambda n: (0,)),
            pl.BlockSpec((_BN, _B), lambda n: (n, 0)),
            pl.BlockSpec((1, 1), lambda n: (0, 0),
                         memory_space=pltpu.SMEM),
            pl.BlockSpec((_B,), lambda n: (0,)),
        ],
        out_shape=[
            jax.ShapeDtypeStruct((_B,), jnp.float32),
            jax.ShapeDtypeStruct((_NI, _B), jnp.float32),
            jax.ShapeDtypeStruct((1, 1), jnp.float32),
            jax.ShapeDtypeStruct((_B,), jnp.float32),
        ],
        scratch_shapes=[pltpu.VMEM((_HID, _B), jnp.float32)],
        compiler_params=pltpu.CompilerParams(
            dimension_semantics=("arbitrary",),
        ),
    )(ue, ie, ub, ib, y, gb2d, eti, item_bias)


def kernel(u, i, y, user_emb, item_emb, user_bias, item_bias, global_bias):
    et = user_emb.T
    eti = item_emb.T
    ue, ie, ub, ib = _sc_emb_gather(et, eti, u, i, user_bias, item_bias)
    gb2d = jnp.reshape(global_bias, (1, 1)).astype(jnp.float32)
    s, score_t, loss, diff = _tc_score(ue, ie, ub, ib, y, gb2d,
                                       eti, item_bias)
    return s, score_t.T, jnp.reshape(loss, ()), diff


# speedup vs baseline: 5.9147x; 1.0029x over previous
import jax
import jax.numpy as jnp
from jax import lax
from jax.experimental import pallas as pl
from jax.experimental.pallas import tpu as pltpu
from jax.experimental.pallas import tpu_sc as plsc

_B = 1024
_HID = 32
_LAMBDA = 1e-05
_NU = 1000000
_NI = 100000

_NC = 2
_NS = 16
_NW = _NC * _NS
_BPW = _B // _NW

_CU_MAX = _NU // 128 - 1
_U_TAIL_OFF = (_CU_MAX + 1) * 128
_U_TAIL_W = _NU - _U_TAIL_OFF
_CI_MAX = _NI // 128 - 1
_I_TAIL_OFF = (_CI_MAX + 1) * 128
_I_TAIL_W = _NI - _I_TAIL_OFF

_RING = 12

_BN = 2048


def _emb_gather_body(et_hbm, eti_hbm, u_hbm, i_hbm, ub_hbm, ib_hbm,
                     ue_out, ie_out, ub_out, ib_out,
                     uidx_v, iidx_v, ue_loc, ie_loc, utail, itail,
                     ub_v, ib_v, bsem,
                     bufs_u, bufs_i, sems_u, sems_i, tsem):
    wid = lax.axis_index("s") * _NC + lax.axis_index("c")
    base = wid * _BPW
    pltpu.sync_copy(u_hbm.at[pl.ds(base, _BPW)], uidx_v)
    pltpu.sync_copy(i_hbm.at[pl.ds(base, _BPW)], iidx_v)
    cb0 = pltpu.async_copy(ub_hbm.at[uidx_v], ub_v, bsem)
    cb1 = pltpu.async_copy(ib_hbm.at[iidx_v], ib_v, bsem)
    ct = pltpu.async_copy(et_hbm.at[:, pl.ds(_U_TAIL_OFF, _U_TAIL_W)],
                          utail, tsem)
    ct.wait()
    ct2 = pltpu.async_copy(eti_hbm.at[:, pl.ds(_I_TAIL_OFF, _I_TAIL_W)],
                           itail, tsem)
    ct2.wait()

    iota16 = lax.iota(jnp.int32, 16)

    def issue(table, idx_v, bufs, sems, cmax, j):
        chunk = idx_v[pl.ds((j // 16) * 16, 16)]
        xj = chunk[j % 16]
        c = jnp.minimum(xj // 128, cmax)
        off = pl.multiple_of(c * 128, 128)
        slot = j % _RING
        return pltpu.async_copy(table.at[:, pl.ds(off, 128)],
                                bufs[slot], sems[slot])

    def extract(idx_v, bufs, tail, loc, cmax, toff, tw, j):
        chunk = idx_v[pl.ds((j // 16) * 16, 16)]
        xj = chunk[j % 16]
        c = jnp.minimum(xj // 128, cmax)
        lane_m = jnp.minimum(xj - c * 128, 127)
        lane_t = jnp.clip(xj - toff, 0, tw - 1)
        in_tail = jnp.full((16,), xj >= toff, jnp.bool_)
        slot = j % _RING
        for h in range(2):
            rows = iota16 + h * 16
            vm = plsc.load_gather(bufs[slot],
                                  [rows, jnp.full((16,), lane_m, jnp.int32)])
            vt = plsc.load_gather(tail,
                                  [rows, jnp.full((16,), lane_t, jnp.int32)])
            loc[j, pl.ds(h * 16, 16)] = jnp.where(in_tail, vt, vm)

    pend_u = [None] * _RING
    pend_i = [None] * _RING
    for j in range(_RING):
        pend_u[j] = issue(et_hbm, uidx_v, bufs_u, sems_u, _CU_MAX, j)
        pend_i[j] = issue(eti_hbm, iidx_v, bufs_i, sems_i, _CI_MAX, j)
    for j in range(_BPW):
        nj = j + _RING
        pend_u[j % _RING].wait()
        extract(uidx_v, bufs_u, utail, ue_loc, _CU_MAX,
                _U_TAIL_OFF, _U_TAIL_W, j)
        if nj < _BPW:
            pend_u[nj % _RING] = issue(et_hbm, uidx_v, bufs_u, sems_u,
                                       _CU_MAX, nj)
        pend_i[j % _RING].wait()
        extract(iidx_v, bufs_i, itail, ie_loc, _CI_MAX,
                _I_TAIL_OFF, _I_TAIL_W, j)
        if nj < _BPW:
            pend_i[nj % _RING] = issue(eti_hbm, iidx_v, bufs_i, sems_i,
                                       _CI_MAX, nj)

    pltpu.sync_copy(ue_loc, ue_out.at[pl.ds(base, _BPW), :])
    pltpu.sync_copy(ie_loc, ie_out.at[pl.ds(base, _BPW), :])
    cb0.wait()
    cb1.wait()
    pltpu.sync_copy(ub_v, ub_out.at[pl.ds(base, _BPW)])
    pltpu.sync_copy(ib_v, ib_out.at[pl.ds(base, _BPW)])


def _sc_emb_gather(et, eti, u, i, user_bias, item_bias):
    mesh = plsc.VectorSubcoreMesh(
        core_axis_name="c", subcore_axis_name="s",
        num_cores=_NC, num_subcores=_NS)
    f = pl.kernel(
        _emb_gather_body,
        out_type=(
            jax.ShapeDtypeStruct((_B, _HID), jnp.float32),
            jax.ShapeDtypeStruct((_B, _HID), jnp.float32),
            jax.ShapeDtypeStruct((_B,), jnp.float32),
            jax.ShapeDtypeStruct((_B,), jnp.float32),
        ),
        mesh=mesh,
        scratch_types=[
            pltpu.VMEM((_BPW,), jnp.int32),
            pltpu.VMEM((_BPW,), jnp.int32),
            pltpu.VMEM((_BPW, _HID), jnp.float32),
            pltpu.VMEM((_BPW, _HID), jnp.float32),
            pltpu.VMEM((_HID, _U_TAIL_W), jnp.float32),
            pltpu.VMEM((_HID, _I_TAIL_W), jnp.float32),
            pltpu.VMEM((_BPW,), jnp.float32),
            pltpu.VMEM((_BPW,), jnp.float32),
            pltpu.SemaphoreType.DMA,
            [pltpu.VMEM((_HID, 128), jnp.float32)] * _RING,
            [pltpu.VMEM((_HID, 128), jnp.float32)] * _RING,
            [pltpu.SemaphoreType.DMA] * _RING,
            [pltpu.SemaphoreType.DMA] * _RING,
            pltpu.SemaphoreType.DMA,
        ],
        compiler_params=pltpu.CompilerParams(use_tc_tiling_on_sc=True,
                                             needs_layout_passes=False),
    )
    return f(et, eti, u, i, user_bias, item_bias)


def _score_body(ue_ref, ie_ref, ub_ref, ib_ref, y_ref, gb_ref,
                eti_ref, ibias_ref,
                s_ref, score_ref, loss_ref, diff_ref,
                at_ref):
    pid = pl.program_id(0)

    @pl.when(pid == 0)
    def _prep():
        at_ref[...] = ue_ref[...].T

    sc = lax.dot_general(eti_ref[...], at_ref[...],
                         (((0,), (0,)), ((), ())),
                         preferred_element_type=jnp.float32)
    score_ref[...] = sc + ibias_ref[...][:, None]

    @pl.when(pid == 0)
    def _small():
        ue = ue_ref[...]
        ie = ie_ref[...]
        ub = ub_ref[...]
        ib = ib_ref[...]
        s = jnp.sum(ue * ie, axis=1) + ub + ib + gb_ref[0, 0]
        d = s - y_ref[...]
        s_ref[...] = s
        diff_ref[...] = d
        l2 = (jnp.mean(ue * ue) + jnp.mean(ie * ie)
              + jnp.mean(ub * ub) + jnp.mean(ib * ib))
        loss_ref[0, 0] = jnp.mean(d * d) + _LAMBDA * l2


def _tc_score(ue, ie, ub, ib, y, gb2d, eti, item_bias):
    grid = (pl.cdiv(_NI, _BN),)
    return pl.pallas_call(
        _score_body,
        grid=grid,
        in_specs=[
            pl.BlockSpec((_B, _HID), lambda n: (0, 0)),
            pl.BlockSpec((_B, _HID), lambda n: (0, 0)),
            pl.BlockSpec((_B,), lambda n: (0,)),
            pl.BlockSpec((_B,), lambda n: (0,)),
            pl.BlockSpec((_B,), lambda n: (0,)),
            pl.BlockSpec((1, 1), lambda n: (0, 0),
                         memory_space=pltpu.SMEM),
            pl.BlockSpec((_HID, _BN), lambda n: (0, n)),
            pl.BlockSpec((_BN,), lambda n: (n,)),
        ],
        out_specs=[
            pl.BlockSpec((_B,), lambda n: (0,)),
            pl.BlockSpec((_BN, _B), lambda n: (n, 0)),
            pl.BlockSpec((1, 1), lambda n: (0, 0),
                         memory_space=pltpu.SMEM),
            pl.BlockSpec((_B,), lambda n: (0,)),
        ],
        out_shape=[
            jax.ShapeDtypeStruct((_B,), jnp.float32),
            jax.ShapeDtypeStruct((_NI, _B), jnp.float32),
            jax.ShapeDtypeStruct((1, 1), jnp.float32),
            jax.ShapeDtypeStruct((_B,), jnp.float32),
        ],
        scratch_shapes=[pltpu.VMEM((_HID, _B), jnp.float32)],
        compiler_params=pltpu.CompilerParams(
            dimension_semantics=("arbitrary",),
        ),
    )(ue, ie, ub, ib, y, gb2d, eti, item_bias)


def kernel(u, i, y, user_emb, item_emb, user_bias, item_bias, global_bias):
    et = user_emb.T
    eti = item_emb.T
    ue, ie, ub, ib = _sc_emb_gather(et, eti, u, i, user_bias, item_bias)
    gb2d = jnp.reshape(global_bias, (1, 1)).astype(jnp.float32)
    s, score_t, loss, diff = _tc_score(ue, ie, ub, ib, y, gb2d,
                                       eti, item_bias)
    return s, score_t.T, jnp.reshape(loss, ()), diff


# speedup vs baseline: 6.0084x; 1.0159x over previous
import jax
import jax.numpy as jnp
from jax import lax
from jax.experimental import pallas as pl
from jax.experimental.pallas import tpu as pltpu
from jax.experimental.pallas import tpu_sc as plsc

_B = 1024
_HID = 32
_LAMBDA = 1e-05
_NU = 1000000
_NI = 100000

_NC = 2
_NS = 16
_NW = _NC * _NS
_BPW = _B // _NW

_RING = 12

_BN = 2048


def _make_gather_body(n_rows):
    cmax = n_rows // 128 - 1
    toff = (cmax + 1) * 128
    tw = n_rows - toff

    def body(et_hbm, idx_hbm, bias_hbm,
             e_out, b_out,
             idx_v, e_loc, tail, b_v, bsem, bufs, sems, tsem):
        wid = lax.axis_index("s") * _NC + lax.axis_index("c")
        base = wid * _BPW
        pltpu.sync_copy(idx_hbm.at[pl.ds(base, _BPW)], idx_v)
        cb = pltpu.async_copy(bias_hbm.at[idx_v], b_v, bsem)
        ctail = pltpu.async_copy(et_hbm.at[:, pl.ds(toff, tw)], tail, tsem)
        ctail.wait()

        iota16 = lax.iota(jnp.int32, 16)

        def issue(j):
            chunk = idx_v[pl.ds((j // 16) * 16, 16)]
            xj = chunk[j % 16]
            c = jnp.minimum(xj // 128, cmax)
            off = pl.multiple_of(c * 128, 128)
            slot = j % _RING
            return pltpu.async_copy(et_hbm.at[:, pl.ds(off, 128)],
                                    bufs[slot], sems[slot])

        def extract(j):
            chunk = idx_v[pl.ds((j // 16) * 16, 16)]
            xj = chunk[j % 16]
            c = jnp.minimum(xj // 128, cmax)
            lane_m = jnp.minimum(xj - c * 128, 127)
            lane_t = jnp.clip(xj - toff, 0, tw - 1)
            in_tail = jnp.full((16,), xj >= toff, jnp.bool_)
            slot = j % _RING
            for h in range(2):
                rows = iota16 + h * 16
                vm = plsc.load_gather(
                    bufs[slot], [rows, jnp.full((16,), lane_m, jnp.int32)])
                vt = plsc.load_gather(
                    tail, [rows, jnp.full((16,), lane_t, jnp.int32)])
                e_loc[j, pl.ds(h * 16, 16)] = jnp.where(in_tail, vt, vm)

        pend = [None] * _RING
        for j in range(_RING):
            pend[j] = issue(j)
        for j in range(_BPW):
            pend[j % _RING].wait()
            extract(j)
            nj = j + _RING
            if nj < _BPW:
                pend[nj % _RING] = issue(nj)

        pltpu.sync_copy(e_loc, e_out.at[pl.ds(base, _BPW), :])
        cb.wait()
        pltpu.sync_copy(b_v, b_out.at[pl.ds(base, _BPW)])

    return body, tw


def _sc_gather(et, idx, bias, n_rows):
    body, tw = _make_gather_body(n_rows)
    mesh = plsc.VectorSubcoreMesh(
        core_axis_name="c", subcore_axis_name="s",
        num_cores=_NC, num_subcores=_NS)
    f = pl.kernel(
        body,
        out_type=(
            jax.ShapeDtypeStruct((_B, _HID), jnp.float32),
            jax.ShapeDtypeStruct((_B,), jnp.float32),
        ),
        mesh=mesh,
        scratch_types=[
            pltpu.VMEM((_BPW,), jnp.int32),
            pltpu.VMEM((_BPW, _HID), jnp.float32),
            pltpu.VMEM((_HID, tw), jnp.float32),
            pltpu.VMEM((_BPW,), jnp.float32),
            pltpu.SemaphoreType.DMA,
            [pltpu.VMEM((_HID, 128), jnp.float32)] * _RING,
            [pltpu.SemaphoreType.DMA] * _RING,
            pltpu.SemaphoreType.DMA,
        ],
        compiler_params=pltpu.CompilerParams(use_tc_tiling_on_sc=True,
                                             needs_layout_passes=False),
    )
    return f(et, idx, bias)


def _score_body(ue_ref, eti_ref, ibias_ref, score_ref, at_ref):
    pid = pl.program_id(0)

    @pl.when(pid == 0)
    def _prep():
        at_ref[...] = ue_ref[...].T

    sc = lax.dot_general(eti_ref[...], at_ref[...],
                         (((0,), (0,)), ((), ())),
                         preferred_element_type=jnp.float32)
    score_ref[...] = sc + ibias_ref[...][:, None]


def _tc_score(ue, eti, item_bias):
    grid = (pl.cdiv(_NI, _BN),)
    return pl.pallas_call(
        _score_body,
        grid=grid,
        in_specs=[
            pl.BlockSpec((_B, _HID), lambda n: (0, 0)),
            pl.BlockSpec((_HID, _BN), lambda n: (0, n)),
            pl.BlockSpec((_BN,), lambda n: (n,)),
        ],
        out_specs=pl.BlockSpec((_BN, _B), lambda n: (n, 0)),
        out_shape=jax.ShapeDtypeStruct((_NI, _B), jnp.float32),
        scratch_shapes=[pltpu.VMEM((_HID, _B), jnp.float32)],
        compiler_params=pltpu.CompilerParams(
            dimension_semantics=("arbitrary",),
        ),
    )(ue, eti, item_bias)


def _small_body(ue_ref, ie_ref, ub_ref, ib_ref, y_ref, gb_ref,
                s_ref, loss_ref, diff_ref):
    ue = ue_ref[...]
    ie = ie_ref[...]
    ub = ub_ref[...]
    ib = ib_ref[...]
    s = jnp.sum(ue * ie, axis=1) + ub + ib + gb_ref[0, 0]
    d = s - y_ref[...]
    s_ref[...] = s
    diff_ref[...] = d
    l2 = (jnp.mean(ue * ue) + jnp.mean(ie * ie)
          + jnp.mean(ub * ub) + jnp.mean(ib * ib))
    loss_ref[0, 0] = jnp.mean(d * d) + _LAMBDA * l2


def _tc_small(ue, ie, ub, ib, y, gb2d):
    return pl.pallas_call(
        _small_body,
        in_specs=[
            pl.BlockSpec((_B, _HID), lambda: (0, 0)),
            pl.BlockSpec((_B, _HID), lambda: (0, 0)),
            pl.BlockSpec((_B,), lambda: (0,)),
            pl.BlockSpec((_B,), lambda: (0,)),
            pl.BlockSpec((_B,), lambda: (0,)),
            pl.BlockSpec((1, 1), lambda: (0, 0), memory_space=pltpu.SMEM),
        ],
        out_specs=[
            pl.BlockSpec((_B,), lambda: (0,)),
            pl.BlockSpec((1, 1), lambda: (0, 0), memory_space=pltpu.SMEM),
            pl.BlockSpec((_B,), lambda: (0,)),
        ],
        out_shape=[
            jax.ShapeDtypeStruct((_B,), jnp.float32),
            jax.ShapeDtypeStruct((1, 1), jnp.float32),
            jax.ShapeDtypeStruct((_B,), jnp.float32),
        ],
    )(ue, ie, ub, ib, y, gb2d)


def kernel(u, i, y, user_emb, item_emb, user_bias, item_bias, global_bias):
    et = user_emb.T
    eti = item_emb.T
    ue, ub = _sc_gather(et, u, user_bias, _NU)
    ie, ib = _sc_gather(eti, i, item_bias, _NI)
    gb2d = jnp.reshape(global_bias, (1, 1)).astype(jnp.float32)
    score_t = _tc_score(ue, eti, item_bias)
    s, loss, diff = _tc_small(ue, ie, ub, ib, y, gb2d)
    return s, score_t.T, jnp.reshape(loss, ()), diff
